# baseline (speedup 1.0000x reference)
import jax
import jax.numpy as jnp
from jax.experimental import pallas as pl


_TOPN = 64
_CW = 2048


def _bits(v):
    return jax.lax.bitcast_convert_type(v, jnp.int32)


def _flt(v):
    return jax.lax.bitcast_convert_type(v, jnp.float32)


def _topk_mask_kernel(x_ref, o_ref):
    B, N = x_ref.shape

    g = jnp.zeros((B, 128), jnp.float32)
    for c0 in range(0, N, _CW):
        ch = x_ref[:, c0:c0 + _CW]
        w = _CW
        while w > 128:
            w //= 2
            ch = jnp.maximum(ch[:, :w], ch[:, w:])
        g = jnp.maximum(g, ch)
    gmax = _bits(jnp.max(g, axis=1, keepdims=True))
    gmin = _bits(jnp.min(g, axis=1, keepdims=True))

    def small_body(_, lohi):
        lo, hi = lohi
        mid = lo + ((hi - lo) >> 1)
        ge = jnp.sum((g >= _flt(mid)).astype(jnp.int32), axis=1,
                     keepdims=True) >= _TOPN
        lo = jnp.where(ge, mid, lo)
        hi = jnp.where(ge, hi, mid)
        return lo, hi

    lo, hi = jax.lax.fori_loop(0, 31, small_body, (gmin, gmax + 1))
    hi = gmax + 1

    def count_ge(midf):
        acc = jnp.zeros((B, 1), jnp.int32)
        for c0 in range(0, N, _CW):
            ch = x_ref[:, c0:c0 + _CW]
            acc = acc + jnp.sum((ch >= midf).astype(jnp.int32), axis=1,
                                keepdims=True)
        return acc

    def p_cond(carry):
        lo, hi = carry
        return jnp.any(hi - lo > 1)

    def p_body(carry):
        lo, hi = carry
        mid = lo + ((hi - lo) >> 1)
        ge = count_ge(_flt(mid)) >= _TOPN
        lo = jnp.where(ge, mid, lo)
        hi = jnp.where(ge, hi, mid)
        return lo, hi

    lo, hi = jax.lax.while_loop(p_cond, p_body, (lo, hi))
    threshf = _flt(lo)

    cnt = count_ge(threshf)
    no_ties = jnp.all(cnt == _TOPN)

    def pos16(c0):
        return (jax.lax.broadcasted_iota(jnp.int16, (B, _CW), 1)
                + jnp.int16(c0))

    def tie_cut(_):
        cnt_gt = jnp.zeros((B, 1), jnp.int32)
        for c0 in range(0, N, _CW):
            ch = x_ref[:, c0:c0 + _CW]
            cnt_gt = cnt_gt + jnp.sum((ch > threshf).astype(jnp.int32),
                                      axis=1, keepdims=True)
        need = _TOPN - cnt_gt
        lo2 = jnp.full((B, 1), -1, jnp.int32)
        hi2 = jnp.full((B, 1), N, jnp.int32)

        def body(_, lohi):
            lo, hi = lohi
            mid = lo + ((hi - lo) >> 1)
            mid16 = (mid - 1).astype(jnp.int16)

            c = jnp.zeros((B, 1), jnp.int32)
            for c0 in range(0, N, _CW):
                ch = x_ref[:, c0:c0 + _CW]
                m = (ch == threshf) & (pos16(c0) <= mid16)
                c = c + jnp.sum(m.astype(jnp.int32), axis=1,
                                keepdims=True)
            ok = c >= need
            return jnp.where(ok, lo, mid), jnp.where(ok, mid, hi)

        _, cut = jax.lax.fori_loop(0, 16, body, (lo2, hi2))
        return cut

    def epilogue_fast(_):
        s = jnp.zeros((B, 1), jnp.float32)
        for c0 in range(0, N, _CW):
            ch = x_ref[:, c0:c0 + _CW]
            s = s + jnp.sum(jnp.where(ch >= threshf, ch, 0.0),
                            axis=1, keepdims=True)
        inv = 1.0 / s
        for c0 in range(0, N, _CW):
            ch = x_ref[:, c0:c0 + _CW]
            o_ref[:, c0:c0 + _CW] = jnp.where(ch >= threshf, ch * inv, 0.0)
        return 0

    def epilogue_ties(_):
        cut16 = (tie_cut(None) - 1).astype(jnp.int16)
        s = jnp.zeros((B, 1), jnp.float32)
        for c0 in range(0, N, _CW):
            ch = x_ref[:, c0:c0 + _CW]
            keep = (ch > threshf) | ((ch == threshf) & (pos16(c0) <= cut16))
            s = s + jnp.sum(jnp.where(keep, ch, 0.0), axis=1,
                            keepdims=True)
        inv = 1.0 / s
        for c0 in range(0, N, _CW):
            ch = x_ref[:, c0:c0 + _CW]
            keep = (ch > threshf) | ((ch == threshf) & (pos16(c0) <= cut16))
            o_ref[:, c0:c0 + _CW] = jnp.where(keep, ch * inv, 0.0)
        return 0

    jax.lax.cond(no_ties, epilogue_fast, epilogue_ties, operand=None)


def kernel(Prob):
    B, N = Prob.shape
    G = 4
    return pl.pallas_call(
        _topk_mask_kernel,
        grid=(G,),
        in_specs=[pl.BlockSpec((B // G, N), lambda i: (i, 0))],
        out_specs=pl.BlockSpec((B // G, N), lambda i: (i, 0)),
        out_shape=jax.ShapeDtypeStruct(Prob.shape, Prob.dtype),
    )(Prob)
